# baseline (speedup 1.0000x reference)
import functools

import jax
import jax.numpy as jnp
from jax import lax
from jax.experimental import pallas as pl
from jax.experimental.pallas import tpu as pltpu
from jax.experimental.pallas import tpu_sc as plsc

_B, _C, _H, _W = 16, 256, 32, 32
_HW = _H * _W
_NC, _NS = 2, 16
_CPT = (2 * _C) // _NS
_BPC = _B // _NC


def _sc_body(row_hbm, col_hbm, out_hbm, tab, blk, sem):
    c = lax.axis_index("c")
    s = lax.axis_index("s")
    is_col = s < (_NS // 2)

    @pl.when(is_col)
    def _():
        pltpu.sync_copy(col_hbm.at[pl.ds(0, _W)], tab)

    @pl.when(jnp.logical_not(is_col))
    def _():
        pltpu.sync_copy(row_hbm.at[pl.ds(0, _H)], tab)

    iota = lax.iota(jnp.int32, 16)
    lane_stride = jnp.where(is_col, 1, _W).astype(jnp.int32)
    step_stride = jnp.where(is_col, _W, 1).astype(jnp.int32)
    ia0 = iota * lane_stride
    ia1 = (iota + 16) * lane_stride
    ci_base = (s % (_NS // 2)) * _CPT

    def cc_body(cc, carry):
        ci = (ci_base + cc).astype(jnp.int32)
        civ = jnp.full((16,), ci, jnp.int32)
        g0 = plsc.load_gather(tab, [iota, civ])
        g1 = plsc.load_gather(tab, [iota + 16, civ])
        ccv = jnp.full((16,), cc, jnp.int32)
        for k in range(_W):
            off = k * step_stride
            plsc.store_scatter(blk, [ccv, ia0 + off], g0)
            plsc.store_scatter(blk, [ccv, ia1 + off], g1)
        return carry

    lax.fori_loop(0, _CPT, cc_body, 0)

    ch0 = s * _CPT
    copies = [
        pltpu.async_copy(blk, out_hbm.at[c * _BPC + i, pl.ds(ch0, _CPT)], sem)
        for i in range(_BPC)
    ]
    for cp in copies:
        cp.wait()


def kernel(mask, row_embed, col_embed):
    B, H, W = mask.shape
    C = row_embed.shape[1]
    mesh = plsc.VectorSubcoreMesh(
        core_axis_name="c", subcore_axis_name="s",
        num_cores=_NC, num_subcores=_NS,
    )
    sc_call = functools.partial(
        pl.kernel,
        out_type=jax.ShapeDtypeStruct((B, 2 * C, H * W), jnp.float32),
        mesh=mesh,
        scratch_types=[
            pltpu.VMEM((_W, _C), jnp.float32),
            pltpu.VMEM((_CPT, _HW), jnp.float32),
            pltpu.SemaphoreType.DMA,
        ],
        compiler_params=pltpu.CompilerParams(
            use_tc_tiling_on_sc=False, needs_layout_passes=False
        ),
    )(_sc_body)
    out = sc_call(row_embed, col_embed)
    return out.reshape(B, 2 * C, H, W)


# speedup vs baseline: 1.7099x; 1.7099x over previous
import functools

import jax
import jax.numpy as jnp
from jax import lax
from jax.experimental import pallas as pl
from jax.experimental.pallas import tpu as pltpu
from jax.experimental.pallas import tpu_sc as plsc

_B, _C, _H, _W = 16, 256, 32, 32
_HW = _H * _W
_NC, _NS = 2, 16
_CPT = (2 * _C) // _NS
_BPC = _B // _NC


def _plane_kernel(row_ref, col_ref, plane_ref):
    col = col_ref[:_W, :]
    row = row_ref[:_H, :]
    k = lax.broadcasted_iota(jnp.int32, (_W, _HW), 1)
    src = lax.broadcasted_iota(jnp.int32, (_W, _HW), 0)
    sel_w = (k % _W == src).astype(jnp.float32)
    sel_h = (k // _W == src).astype(jnp.float32)
    dn = (((0,), (0,)), ((), ()))
    plane_ref[:_C] = lax.dot_general(col, sel_w, dn, preferred_element_type=jnp.float32)
    plane_ref[_C:] = lax.dot_general(row, sel_h, dn, preferred_element_type=jnp.float32)


def _fanout_body(plane_hbm, out_hbm, blk, sem):
    c = lax.axis_index("c")
    s = lax.axis_index("s")
    ch0 = s * _CPT
    pltpu.sync_copy(plane_hbm.at[pl.ds(ch0, _CPT)], blk)
    copies = [
        pltpu.async_copy(blk, out_hbm.at[c * _BPC + i, pl.ds(ch0, _CPT)], sem)
        for i in range(_BPC)
    ]
    for cp in copies:
        cp.wait()


def kernel(mask, row_embed, col_embed):
    B, H, W = mask.shape
    C = row_embed.shape[1]
    plane = pl.pallas_call(
        _plane_kernel,
        in_specs=[
            pl.BlockSpec(memory_space=pltpu.VMEM),
            pl.BlockSpec(memory_space=pltpu.VMEM),
        ],
        out_specs=pl.BlockSpec(memory_space=pltpu.VMEM),
        out_shape=jax.ShapeDtypeStruct((2 * C, H * W), jnp.float32),
    )(row_embed, col_embed)

    mesh = plsc.VectorSubcoreMesh(
        core_axis_name="c", subcore_axis_name="s",
        num_cores=_NC, num_subcores=_NS,
    )
    fanout = functools.partial(
        pl.kernel,
        out_type=jax.ShapeDtypeStruct((B, 2 * C, H * W), jnp.float32),
        mesh=mesh,
        scratch_types=[
            pltpu.VMEM((_CPT, _HW), jnp.float32),
            pltpu.SemaphoreType.DMA,
        ],
    )(_fanout_body)
    out = fanout(plane)
    return out.reshape(B, 2 * C, H, W)


# speedup vs baseline: 3.4821x; 2.0365x over previous
import functools

import jax
import jax.numpy as jnp
from jax import lax
from jax.experimental import pallas as pl
from jax.experimental.pallas import tpu as pltpu
from jax.experimental.pallas import tpu_sc as plsc

_B, _C, _H, _W = 16, 256, 32, 32
_HW = _H * _W
_NC, _NS = 2, 16
_RPT = _HW // _NS
_BPC = _B // _NC


def _plane_kernel(row_ref, col_ref, plane_ref):
    col = col_ref[:_W, :]
    row = row_ref[:_H, :]
    plane_ref[:, :_C] = jnp.broadcast_to(
        col[None, :, :], (_H, _W, _C)).reshape(_HW, _C)
    plane_ref[:, _C:] = jnp.broadcast_to(
        row[:, None, :], (_H, _W, _C)).reshape(_HW, _C)


def _fanout_body(plane_hbm, out_hbm, blk, sem):
    c = lax.axis_index("c")
    s = lax.axis_index("s")
    r0 = s * _RPT
    pltpu.sync_copy(plane_hbm.at[pl.ds(r0, _RPT)], blk)
    copies = [
        pltpu.async_copy(blk, out_hbm.at[c * _BPC + i, pl.ds(r0, _RPT)], sem)
        for i in range(_BPC)
    ]
    for cp in copies:
        cp.wait()


def kernel(mask, row_embed, col_embed):
    B, H, W = mask.shape
    C = row_embed.shape[1]
    plane = pl.pallas_call(
        _plane_kernel,
        in_specs=[
            pl.BlockSpec(memory_space=pltpu.VMEM),
            pl.BlockSpec(memory_space=pltpu.VMEM),
        ],
        out_specs=pl.BlockSpec(memory_space=pltpu.VMEM),
        out_shape=jax.ShapeDtypeStruct((H * W, 2 * C), jnp.float32),
    )(row_embed, col_embed)

    mesh = plsc.VectorSubcoreMesh(
        core_axis_name="c", subcore_axis_name="s",
        num_cores=_NC, num_subcores=_NS,
    )
    fanout = functools.partial(
        pl.kernel,
        out_type=jax.ShapeDtypeStruct((B, H * W, 2 * C), jnp.float32),
        mesh=mesh,
        scratch_types=[
            pltpu.VMEM((_RPT, 2 * _C), jnp.float32),
            pltpu.SemaphoreType.DMA,
        ],
    )(_fanout_body)
    out = fanout(plane)
    return out.reshape(B, H, W, 2 * C).transpose(0, 3, 1, 2)
